# baseline (speedup 1.0000x reference)
import jax
import jax.numpy as jnp
from jax.experimental import pallas as pl
from jax.experimental.pallas import tpu as pltpu


def _prep_body(a_ref, x_ref, w1_ref, o_r_ref, a16_ref, xs1_ref):
    a = a_ref[...]
    s = jnp.sum(a, axis=1, keepdims=True)
    r = jax.lax.rsqrt(s + 1.0)
    o_r_ref[...] = r
    a16_ref[...] = a.astype(jnp.bfloat16)
    f = jnp.dot(x_ref[...].astype(jnp.bfloat16), w1_ref[...],
                preferred_element_type=jnp.float32)
    xs1_ref[...] = (r * f).astype(jnp.bfloat16)


def _make_agg_body(rows):
    def _agg_body(a_ref, xs1_ref, r_ref, b1_ref, b2_ref, al_ref, w2_ref,
                  o_ref, xs2_scr):
        p = pl.program_id(0)
        i = pl.program_id(1)
        sl = pl.ds(i * rows, rows)

        @pl.when(p == 0)
        def _layer1():
            acc = jnp.dot(a_ref[...], xs1_ref[...],
                          preferred_element_type=jnp.float32)
            r = r_ref[sl, :]
            t = r * (acc + xs1_ref[sl, :].astype(jnp.float32)) + b1_ref[...]
            t = jnp.where(t >= 0.0, t, t * al_ref[...])
            f2 = jnp.dot(t.astype(jnp.bfloat16), w2_ref[...],
                         preferred_element_type=jnp.float32)
            xs2_scr[sl, :] = (r * f2).astype(jnp.bfloat16)

        @pl.when(p == 1)
        def _layer2():
            acc = jnp.dot(a_ref[...], xs2_scr[...],
                          preferred_element_type=jnp.float32)
            t = (r_ref[sl, :] * (acc + xs2_scr[sl, :].astype(jnp.float32))
                 + b2_ref[...])
            o_ref[...] = jnp.where(t >= 0.0, t, t * al_ref[...])

    return _agg_body


def _pick_block(n, cands):
    for rb in cands:
        if n % rb == 0:
            return rb
    return n


def kernel(seq, adj, W1, W2, bias1, bias2, prelu_a):
    n = adj.shape[0]
    f1 = W1.shape[1]
    f2 = W2.shape[1]
    b1 = bias1.reshape(1, -1)
    b2 = bias2.reshape(1, -1)
    a2d = prelu_a.reshape(1, 1)
    w1b = W1.astype(jnp.bfloat16)
    w2b = W2.astype(jnp.bfloat16)

    rb = _pick_block(n, (400, 200, 80, 40, 8))
    r_inv, adj16, xs1 = pl.pallas_call(
        _prep_body,
        grid=(n // rb,),
        in_specs=[
            pl.BlockSpec((rb, n), lambda i: (i, 0)),
            pl.BlockSpec((rb, seq.shape[1]), lambda i: (i, 0)),
            pl.BlockSpec(w1b.shape, lambda i: (0, 0)),
        ],
        out_specs=[
            pl.BlockSpec((rb, 1), lambda i: (i, 0)),
            pl.BlockSpec((rb, n), lambda i: (i, 0)),
            pl.BlockSpec((rb, f1), lambda i: (i, 0)),
        ],
        out_shape=[
            jax.ShapeDtypeStruct((n, 1), jnp.float32),
            jax.ShapeDtypeStruct((n, n), jnp.bfloat16),
            jax.ShapeDtypeStruct((n, f1), jnp.bfloat16),
        ],
        compiler_params=pltpu.CompilerParams(
            dimension_semantics=("parallel",)),
    )(adj, seq, w1b)

    rba = _pick_block(n, (1000, 400, 200, 80, 40, 8))
    out2 = pl.pallas_call(
        _make_agg_body(rba),
        grid=(2, n // rba),
        in_specs=[
            pl.BlockSpec((rba, n), lambda p, i: (i, 0)),
            pl.BlockSpec((n, f1), lambda p, i: (0, 0)),
            pl.BlockSpec((n, 1), lambda p, i: (0, 0)),
            pl.BlockSpec((1, f1), lambda p, i: (0, 0)),
            pl.BlockSpec((1, f2), lambda p, i: (0, 0)),
            pl.BlockSpec((1, 1), lambda p, i: (0, 0)),
            pl.BlockSpec(w2b.shape, lambda p, i: (0, 0)),
        ],
        out_specs=pl.BlockSpec((rba, f2), lambda p, i: (p * i, 0)),
        out_shape=jax.ShapeDtypeStruct((n, f2), jnp.float32),
        scratch_shapes=[pltpu.VMEM((n, f2), jnp.bfloat16)],
        compiler_params=pltpu.CompilerParams(
            dimension_semantics=("arbitrary", "arbitrary")),
    )(adj16, xs1, r_inv, b1, b2, a2d, w2b)
    return out2


# speedup vs baseline: 1.0007x; 1.0007x over previous
import jax
import jax.numpy as jnp
from jax.experimental import pallas as pl
from jax.experimental.pallas import tpu as pltpu


def _prep_body(a_ref, x_ref, w1_ref, o_r_ref, a16_ref, xs1_ref):
    a = a_ref[...]
    s = jnp.sum(a, axis=1, keepdims=True)
    r = jax.lax.rsqrt(s + 1.0)
    o_r_ref[...] = r
    a16_ref[...] = a.astype(jnp.bfloat16)
    f = jnp.dot(x_ref[...].astype(jnp.bfloat16), w1_ref[...],
                preferred_element_type=jnp.float32)
    xs1_ref[...] = (r * f).astype(jnp.bfloat16)


def _make_agg_body(rows):
    def _agg_body(a_ref, xs1_ref, r_ref, b1_ref, b2_ref, al_ref, w2_ref,
                  o_ref, xs2_scr):
        p = pl.program_id(0)
        i = pl.program_id(1)
        sl = pl.ds(i * rows, rows)

        @pl.when(p == 0)
        def _layer1():
            acc = jnp.dot(a_ref[...], xs1_ref[...],
                          preferred_element_type=jnp.float32)
            r = r_ref[sl, :]
            t = r * (acc + xs1_ref[sl, :].astype(jnp.float32)) + b1_ref[...]
            t = jnp.where(t >= 0.0, t, t * al_ref[...])
            f2 = jnp.dot(t.astype(jnp.bfloat16), w2_ref[...],
                         preferred_element_type=jnp.float32)
            xs2_scr[sl, :] = (r * f2).astype(jnp.bfloat16)

        @pl.when(p == 1)
        def _layer2():
            acc = jnp.dot(a_ref[...], xs2_scr[...],
                          preferred_element_type=jnp.float32)
            t = (r_ref[sl, :] * (acc + xs2_scr[sl, :].astype(jnp.float32))
                 + b2_ref[...])
            o_ref[...] = jnp.where(t >= 0.0, t, t * al_ref[...])

    return _agg_body


def _pick_block(n, cands):
    for rb in cands:
        if n % rb == 0:
            return rb
    return n


def kernel(seq, adj, W1, W2, bias1, bias2, prelu_a):
    n = adj.shape[0]
    f1 = W1.shape[1]
    f2 = W2.shape[1]
    b1 = bias1.reshape(1, -1)
    b2 = bias2.reshape(1, -1)
    a2d = prelu_a.reshape(1, 1)
    w1b = W1.astype(jnp.bfloat16)
    w2b = W2.astype(jnp.bfloat16)

    rb = _pick_block(n, (200, 80, 40, 8))
    r_inv, adj16, xs1 = pl.pallas_call(
        _prep_body,
        grid=(n // rb,),
        in_specs=[
            pl.BlockSpec((rb, n), lambda i: (i, 0)),
            pl.BlockSpec((rb, seq.shape[1]), lambda i: (i, 0)),
            pl.BlockSpec(w1b.shape, lambda i: (0, 0)),
        ],
        out_specs=[
            pl.BlockSpec((rb, 1), lambda i: (i, 0)),
            pl.BlockSpec((rb, n), lambda i: (i, 0)),
            pl.BlockSpec((rb, f1), lambda i: (i, 0)),
        ],
        out_shape=[
            jax.ShapeDtypeStruct((n, 1), jnp.float32),
            jax.ShapeDtypeStruct((n, n), jnp.bfloat16),
            jax.ShapeDtypeStruct((n, f1), jnp.bfloat16),
        ],
        compiler_params=pltpu.CompilerParams(
            dimension_semantics=("parallel",)),
    )(adj, seq, w1b)

    rba = _pick_block(n, (1000, 400, 200, 80, 40, 8))
    out2 = pl.pallas_call(
        _make_agg_body(rba),
        grid=(2, n // rba),
        in_specs=[
            pl.BlockSpec((rba, n), lambda p, i: (i, 0)),
            pl.BlockSpec((n, f1), lambda p, i: (0, 0)),
            pl.BlockSpec((n, 1), lambda p, i: (0, 0)),
            pl.BlockSpec((1, f1), lambda p, i: (0, 0)),
            pl.BlockSpec((1, f2), lambda p, i: (0, 0)),
            pl.BlockSpec((1, 1), lambda p, i: (0, 0)),
            pl.BlockSpec(w2b.shape, lambda p, i: (0, 0)),
        ],
        out_specs=pl.BlockSpec((rba, f2), lambda p, i: (p * i, 0)),
        out_shape=jax.ShapeDtypeStruct((n, f2), jnp.float32),
        scratch_shapes=[pltpu.VMEM((n, f2), jnp.bfloat16)],
        compiler_params=pltpu.CompilerParams(
            dimension_semantics=("arbitrary", "arbitrary")),
    )(adj16, xs1, r_inv, b1, b2, a2d, w2b)
    return out2
